# Initial kernel scaffold; baseline (speedup 1.0000x reference)
import jax
import jax.numpy as jnp
from jax.experimental import pallas as pl


def kernel(input_tensor, indices):
    raise NotImplementedError("write your pallas kernel here")


# speedup vs baseline: 2.7643x; 2.7643x over previous
import functools

import jax
import jax.numpy as jnp
from jax import lax
from jax.experimental import pallas as pl
from jax.experimental.pallas import tpu as pltpu
from jax.experimental.pallas import tpu_sc as plsc

D = 128
CHUNK = 128
NC = 2
NS = 16
NW = NC * NS


@functools.lru_cache(maxsize=None)
def _build_gather(B, V, d):
    b_per_w = B // NW
    nchunk = b_per_w // CHUNK
    mesh = plsc.VectorSubcoreMesh(core_axis_name="c", subcore_axis_name="s")

    @functools.partial(
        pl.kernel,
        mesh=mesh,
        out_type=jax.ShapeDtypeStruct((B, d), jnp.float32),
        scratch_types=[
            pltpu.VMEM((CHUNK,), jnp.int32),
            pltpu.VMEM((CHUNK, d), jnp.float32),
            pltpu.SemaphoreType.DMA,
        ],
    )
    def k(table_hbm, idx_hbm, out_hbm, idx_v, rows_v, sem):
        wid = lax.axis_index("s") * NC + lax.axis_index("c")
        base = wid * b_per_w

        def body(c, carry):
            off = base + c * CHUNK
            pltpu.sync_copy(idx_hbm.at[pl.ds(off, CHUNK)], idx_v)
            pltpu.async_copy(table_hbm.at[idx_v], rows_v, sem).wait()
            pltpu.sync_copy(rows_v, out_hbm.at[pl.ds(off, CHUNK)])
            return carry

        lax.fori_loop(0, nchunk, body, 0)

    return k


def kernel(input_tensor, indices):
    d = input_tensor.shape[-1]
    idx = indices.reshape(-1).astype(jnp.int32)
    out = _build_gather(idx.shape[0], input_tensor.shape[0], d)(input_tensor, idx)
    return out.reshape(indices.shape + (d,))


# speedup vs baseline: 3.1257x; 1.1307x over previous
import functools

import jax
import jax.numpy as jnp
from jax import lax
from jax.experimental import pallas as pl
from jax.experimental.pallas import tpu as pltpu
from jax.experimental.pallas import tpu_sc as plsc

CHUNK = 128
NBUF = 2
NC = 2
NS = 16
NW = NC * NS


@functools.lru_cache(maxsize=None)
def _build_gather(B, V, d):
    b_per_w = B // NW
    nchunk = b_per_w // CHUNK
    ngroups = nchunk // NBUF
    assert b_per_w * NW == B and nchunk * CHUNK == b_per_w
    assert ngroups * NBUF == nchunk
    mesh = plsc.VectorSubcoreMesh(core_axis_name="c", subcore_axis_name="s")

    @functools.partial(
        pl.kernel,
        mesh=mesh,
        out_type=jax.ShapeDtypeStruct((B, d), jnp.float32),
        scratch_types=[
            pltpu.VMEM((b_per_w,), jnp.int32),
            pltpu.VMEM((NBUF, CHUNK, d), jnp.float32),
            pltpu.SemaphoreType.DMA,
        ] + [pltpu.SemaphoreType.DMA] * NBUF,
    )
    def k(table_hbm, idx_hbm, out_hbm, idx_v, rows_v, sem_g, *sems_o):
        wid = lax.axis_index("s") * NC + lax.axis_index("c")
        base = wid * b_per_w
        pltpu.sync_copy(idx_hbm.at[pl.ds(base, b_per_w)], idx_v)

        def out_desc(c, b):
            return pltpu.make_async_copy(
                rows_v.at[b], out_hbm.at[pl.ds(base + c * CHUNK, CHUNK)], sems_o[b])

        def gather(c, b):
            pltpu.async_copy(
                table_hbm.at[idx_v.at[pl.ds(c * CHUNK, CHUNK)]],
                rows_v.at[b], sem_g).wait()

        for b in range(NBUF):
            gather(b, b)
            out_desc(b, b).start()

        def group(g, carry):
            for b in range(NBUF):
                c = g * NBUF + b
                out_desc(c - NBUF, b).wait()
                gather(c, b)
                out_desc(c, b).start()
            return carry

        lax.fori_loop(1, ngroups, group, 0)

        for b in range(NBUF):
            out_desc((ngroups - 1) * NBUF + b, b).wait()

    return k


def kernel(input_tensor, indices):
    d = input_tensor.shape[-1]
    B = indices.size
    idx = indices.reshape(-1).astype(jnp.int32)
    out = _build_gather(B, input_tensor.shape[0], d)(input_tensor, idx)
    return out.reshape(indices.shape + (d,))


# speedup vs baseline: 3.3425x; 1.0694x over previous
import functools

import jax
import jax.numpy as jnp
from jax import lax
from jax.experimental import pallas as pl
from jax.experimental.pallas import tpu as pltpu
from jax.experimental.pallas import tpu_sc as plsc

CHUNK = 128
NBUF = 5
SKEW = 2
NC = 2
NS = 16
NW = NC * NS


@functools.lru_cache(maxsize=None)
def _build_gather(B, V, d):
    b_per_w = B // NW
    nchunk = b_per_w // CHUNK
    ngroups = nchunk // NBUF
    assert b_per_w * NW == B and nchunk * CHUNK == b_per_w
    assert ngroups * NBUF == nchunk
    mesh = plsc.VectorSubcoreMesh(core_axis_name="c", subcore_axis_name="s")

    @functools.partial(
        pl.kernel,
        mesh=mesh,
        out_type=jax.ShapeDtypeStruct((B, d), jnp.float32),
        scratch_types=[
            pltpu.VMEM((b_per_w,), jnp.int32),
            pltpu.VMEM((NBUF, CHUNK, d), jnp.float32),
        ] + [pltpu.SemaphoreType.DMA] * (2 * NBUF),
    )
    def k(table_hbm, idx_hbm, out_hbm, idx_v, rows_v, *sems):
        sems_g, sems_o = sems[:NBUF], sems[NBUF:]
        wid = lax.axis_index("s") * NC + lax.axis_index("c")
        base = wid * b_per_w
        pltpu.sync_copy(idx_hbm.at[pl.ds(base, b_per_w)], idx_v)

        def out_desc(c, b):
            return pltpu.make_async_copy(
                rows_v.at[b], out_hbm.at[pl.ds(base + c * CHUNK, CHUNK)], sems_o[b])

        def gather_desc(c, b):
            return pltpu.make_async_copy(
                table_hbm.at[idx_v.at[pl.ds(c * CHUNK, CHUNK)]],
                rows_v.at[b], sems_g[b])

        def drain(c, b):
            gather_desc(c, b).wait()
            out_desc(c, b).start()

        for b in range(NBUF):
            gather_desc(b, b).start()
            if b >= SKEW:
                drain(b - SKEW, b - SKEW)

        def group(g, carry):
            for b in range(NBUF):
                c = g * NBUF + b
                out_desc(c - NBUF, b).wait()
                gather_desc(c, b).start()
                drain(c - SKEW, (b - SKEW) % NBUF)
            return carry

        lax.fori_loop(1, ngroups, group, 0)

        for c in range(nchunk - SKEW, nchunk):
            drain(c, c % NBUF)
        for b in range(NBUF):
            out_desc((ngroups - 1) * NBUF + b, b).wait()

    return k


def kernel(input_tensor, indices):
    d = input_tensor.shape[-1]
    B = indices.size
    idx = indices.reshape(-1).astype(jnp.int32)
    out = _build_gather(B, input_tensor.shape[0], d)(input_tensor, idx)
    return out.reshape(indices.shape + (d,))
